# baseline (speedup 1.0000x reference)
import jax
import jax.numpy as jnp
from jax.experimental import pallas as pl

_N = 16384
_C = 1000
_R = 256
_NB = _N // _R


def _body(y_ref, x_ref, o_ref):
    y = y_ref[0, 0, :]
    cols = jax.lax.broadcasted_iota(jnp.int32, (_R, _C), 1)
    eq = cols == y[:, None]
    o_ref[0, 0, :] = jnp.sum(jnp.where(eq, x_ref[...], 0.0), axis=1)


def kernel(logits, y):
    y2 = y.astype(jnp.int32).reshape(_NB, 1, _R)
    out = pl.pallas_call(
        _body,
        grid=(_NB,),
        in_specs=[
            pl.BlockSpec((1, 1, _R), lambda i: (i, 0, 0)),
            pl.BlockSpec((_R, _C), lambda i: (i, 0)),
        ],
        out_specs=pl.BlockSpec((1, 1, _R), lambda i: (i, 0, 0)),
        out_shape=jax.ShapeDtypeStruct((_NB, 1, _R), jnp.float32),
    )(y2, logits)
    return out.reshape(-1)
